# Initial kernel scaffold; baseline (speedup 1.0000x reference)
import jax
import jax.numpy as jnp
from jax.experimental import pallas as pl


def kernel(feat, efeat, W_attn, b_attn, W_e1, b_e1, W_e2, b_e2, bias, edge_index, etype):
    raise NotImplementedError("write your pallas kernel here")


# speedup vs baseline: 74.3034x; 74.3034x over previous
import jax
import jax.numpy as jnp
from jax import lax
from jax.experimental import pallas as pl
from jax.experimental.pallas import tpu as pltpu
from jax.experimental.pallas import tpu_sc as plsc

N_NODES = 10000
N_EDGES = 160000
F = 16
NEG_SLOPE = 0.01

NC, NS = 2, 16
NW = NC * NS
EW = N_EDGES // NW
CH = 128
NFULL = EW // CH
TAIL = EW - NFULL * CH
ROWS2 = 2 * N_NODES
DUMP = ROWS2
SROWS = ROWS2 + 96
ZPT = SROWS // NS
WPT = 1248
PAYW = 2 * F

FPT = 624
FLAST = N_NODES - 15 * FPT

_CHUNKS = [(i * CH, CH) for i in range(NFULL)] + ([(NFULL * CH, TAIL)] if TAIL else [])

_mesh = plsc.VectorSubcoreMesh(
    core_axis_name="c", subcore_axis_name="s", num_cores=NC, num_subcores=NS)


def _gather_body(feat_h, src_h, dst_h, hsrc_h, zdst_h,
                 idxs_v, idxd_v, rs0, rs1, rd0, rd1,
                 gs0, gs1, gd0, gd1, ss0, ss1, sd0, sd1):
  c = lax.axis_index("c")
  s = lax.axis_index("s")
  base = (c * NS + s) * EW

  pltpu.sync_copy(src_h.at[pl.ds(base, EW)], idxs_v)
  pltpu.sync_copy(dst_h.at[pl.ds(base, EW)], idxd_v)

  rs = (rs0, rs1)
  rd = (rd0, rd1)
  gsem = ((gs0, gs1), (gd0, gd1))
  ssem = ((ss0, ss1), (sd0, sd1))
  n = len(_CHUNKS)
  g = [[None, None], [None, None]]
  st = [[None, None], [None, None]]

  def start_gather(i):
    b = i & 1
    off, sz = _CHUNKS[i]
    g[0][b] = pltpu.async_copy(
        feat_h.at[idxs_v.at[pl.ds(off, sz)]], rs[b].at[pl.ds(0, sz)], gsem[0][b])
    g[1][b] = pltpu.async_copy(
        feat_h.at[idxd_v.at[pl.ds(off, sz)]], rd[b].at[pl.ds(0, sz)], gsem[1][b])

  start_gather(0)
  for i in range(n):
    b = i & 1
    g[0][b].wait()
    g[1][b].wait()
    if i + 1 < n:
      if st[0][1 - b] is not None:
        st[0][1 - b].wait()
        st[1][1 - b].wait()
      start_gather(i + 1)
    off, sz = _CHUNKS[i]
    st[0][b] = pltpu.async_copy(
        rs[b].at[pl.ds(0, sz)], hsrc_h.at[pl.ds(base + off, sz)], ssem[0][b])
    st[1][b] = pltpu.async_copy(
        rd[b].at[pl.ds(0, sz)], zdst_h.at[pl.ds(base + off, sz)], ssem[1][b])
  for j in range(2):
    for b in range(2):
      if st[j][b] is not None:
        st[j][b].wait()


def _sc_gather(feat, src, dst):
  return pl.kernel(
      _gather_body,
      out_type=[jax.ShapeDtypeStruct((N_EDGES, F), jnp.float32),
                jax.ShapeDtypeStruct((N_EDGES, F), jnp.float32)],
      mesh=_mesh,
      compiler_params=pltpu.CompilerParams(use_tc_tiling_on_sc=False),
      scratch_types=[
          pltpu.VMEM((EW,), jnp.int32),
          pltpu.VMEM((EW,), jnp.int32),
          pltpu.VMEM((CH, F), jnp.float32),
          pltpu.VMEM((CH, F), jnp.float32),
          pltpu.VMEM((CH, F), jnp.float32),
          pltpu.VMEM((CH, F), jnp.float32),
      ] + [pltpu.SemaphoreType.DMA] * 8,
  )(feat, src, dst)


BK2 = 1000


def _k2_body(h_ref, z_ref, ef_ref, et_ref, was_ref, wad_ref, ba_ref,
             wcat_ref, bcat_ref, out_ref):
  h = h_ref[...]
  z = z_ref[...]
  ef = ef_ref[...]
  s1 = jnp.sum(h * was_ref[...], axis=1, keepdims=True)
  s2 = jnp.sum(z * wad_ref[...], axis=1, keepdims=True)
  a = s1 + s2 + ba_ref[...]
  a = jnp.where(a >= 0.0, a, NEG_SLOPE * a)
  ex = jnp.exp(a)
  col = lax.broadcasted_iota(jnp.int32, (F, F * F), 1)
  row = lax.broadcasted_iota(jnp.int32, (F, F * F), 0)
  rm = (col // F == row).astype(jnp.float32)
  tm = (col % F == row).astype(jnp.float32)
  p = (jnp.dot(ef, rm, preferred_element_type=jnp.float32) *
       jnp.dot(h, tm, preferred_element_type=jnp.float32))
  v = (jnp.dot(p, wcat_ref[...], preferred_element_type=jnp.float32) +
       jnp.dot(h, bcat_ref[...], preferred_element_type=jnp.float32))
  vsel = jnp.where(et_ref[...] == 0, v[:, :F], v[:, F:])
  out_ref[...] = jnp.concatenate(
      [ex * vsel, ex, jnp.zeros((BK2, F - 1), jnp.float32)], axis=1)


def _tc_dense(hsrc, zdst, efeat, et2, was, wad, ba, wcat, bcat):
  grid = (N_EDGES // BK2,)
  return pl.pallas_call(
      _k2_body,
      grid=grid,
      in_specs=[
          pl.BlockSpec((BK2, F), lambda i: (i, 0)),
          pl.BlockSpec((BK2, F), lambda i: (i, 0)),
          pl.BlockSpec((BK2, F), lambda i: (i, 0)),
          pl.BlockSpec((BK2, 1), lambda i: (i, 0)),
          pl.BlockSpec((1, F), lambda i: (0, 0)),
          pl.BlockSpec((1, F), lambda i: (0, 0)),
          pl.BlockSpec((1, 1), lambda i: (0, 0)),
          pl.BlockSpec((F * F, 2 * F), lambda i: (0, 0)),
          pl.BlockSpec((F, 2 * F), lambda i: (0, 0)),
      ],
      out_specs=pl.BlockSpec((BK2, PAYW), lambda i: (i, 0)),
      out_shape=jax.ShapeDtypeStruct((N_EDGES, PAYW), jnp.float32),
  )(hsrc, zdst, efeat, et2, was, wad, ba, wcat, bcat)


def _scatter_body(pay_h, dst_h, et_h, spart_h,
                  idxd_v, idxt_v, r0, r1, rt, p0, p1, sacc,
                  ps0, ps1, cs0, cs1):
  c = lax.axis_index("c")
  s = lax.axis_index("s")
  base = (c * NS + s) * EW
  for i in range(CH):
    p0[i, pl.ds(0, 16)] = jnp.zeros((16,), jnp.float32)
    p0[i, pl.ds(16, 16)] = jnp.zeros((16,), jnp.float32)
  for j in range(9):
    pltpu.sync_copy(p0.at[pl.ds(0, CH)],
                    sacc.at[pl.ds(s * ZPT + j * CH, CH)])
  pltpu.sync_copy(p0.at[pl.ds(0, ZPT - 9 * CH)],
                  sacc.at[pl.ds(s * ZPT + 9 * CH, ZPT - 9 * CH)])

  plsc.subcore_barrier()

  pltpu.sync_copy(dst_h.at[pl.ds(base, EW)], idxd_v.at[pl.ds(0, EW)])
  pltpu.sync_copy(et_h.at[pl.ds(base, EW)], idxt_v.at[pl.ds(0, EW)])

  rbuf = (r0, r1)
  pbuf = (p0, p1)
  psem = (ps0, ps1)
  csem = (cs0, cs1)
  n = len(_CHUNKS)
  pd = [None, None]
  cd = [None, None]

  def compute_ridx(i, rb):
    off = i * CH
    for k in range(CH // 16):
      d16 = idxd_v[pl.ds(off + k * 16, 16)]
      t16 = idxt_v[pl.ds(off + k * 16, 16)]
      rb[pl.ds(k * 16, 16)] = d16 + t16 * N_NODES

  def compute_tail_ridx():
    d16 = idxd_v[pl.ds(NFULL * CH, 16)]
    t16 = idxt_v[pl.ds(NFULL * CH, 16)]
    lane = lax.iota(jnp.int32, 16)
    rt[...] = jnp.where(lane < TAIL, d16 + t16 * N_NODES, DUMP)

  def start_pay(i):
    b = i & 1
    off, sz = _CHUNKS[i]
    pd[b] = pltpu.async_copy(
        pay_h.at[pl.ds(base + off, sz)], pbuf[b].at[pl.ds(0, sz)], psem[b])

  start_pay(0)
  compute_ridx(0, r0)
  for i in range(n):
    b = i & 1
    if i + 1 < n:
      if cd[1 - b] is not None:
        cd[1 - b].wait()
      start_pay(i + 1)
      if TAIL and i + 1 == n - 1:
        compute_tail_ridx()
      else:
        compute_ridx(i + 1, rbuf[1 - b])
    pd[b].wait()
    _, sz = _CHUNKS[i]
    if sz == CH:
      cd[b] = pltpu.async_copy(
          pbuf[b].at[pl.ds(0, CH)], sacc.at[rbuf[b]], csem[b], add=True)
    else:
      cd[b] = pltpu.async_copy(
          pbuf[b].at[pl.ds(0, 16)], sacc.at[rt], csem[b], add=True)
  for b in range(2):
    if cd[b] is not None:
      cd[b].wait()

  plsc.subcore_barrier()

  @pl.when(s < NS - 1)
  def _wb():
    pltpu.sync_copy(sacc.at[pl.ds(s * WPT, WPT)],
                    spart_h.at[c, pl.ds(s * WPT, WPT)])

  @pl.when(s == NS - 1)
  def _wb_last():
    pltpu.sync_copy(sacc.at[pl.ds(15 * WPT, ROWS2 - 15 * WPT)],
                    spart_h.at[c, pl.ds(15 * WPT, ROWS2 - 15 * WPT)])


def _sc_scatter(payload, dst, et):
  return pl.kernel(
      _scatter_body,
      out_type=jax.ShapeDtypeStruct((NC, ROWS2, PAYW), jnp.float32),
      mesh=_mesh,
      compiler_params=pltpu.CompilerParams(use_tc_tiling_on_sc=False),
      scratch_types=[
          pltpu.VMEM((EW + 16 - TAIL,), jnp.int32),
          pltpu.VMEM((EW + 16 - TAIL,), jnp.int32),
          pltpu.VMEM((CH,), jnp.int32),
          pltpu.VMEM((CH,), jnp.int32),
          pltpu.VMEM((16,), jnp.int32),
          pltpu.VMEM((CH, PAYW), jnp.float32),
          pltpu.VMEM((CH, PAYW), jnp.float32),
          pltpu.VMEM_SHARED((SROWS, PAYW), jnp.float32),
      ] + [pltpu.SemaphoreType.DMA] * 4,
  )(payload, dst, et)


def _k4_body(sp0_ref, sp1_ref, bias_ref, out_ref):
  x0 = sp0_ref[0] + sp0_ref[1]
  x1 = sp1_ref[0] + sp1_ref[1]
  m0 = x0[:, 0:F]
  d0 = x0[:, F:F + 1]
  m1 = x1[:, 0:F]
  d1 = x1[:, F:F + 1]
  out_ref[...] = (m0 / jnp.where(d0 > 0, d0, 1.0) +
                  m1 / jnp.where(d1 > 0, d1, 1.0) + bias_ref[...])


def _tc_finalize(spart, bias2):
  return pl.pallas_call(
      _k4_body,
      grid=(1,),
      in_specs=[pl.BlockSpec((NC, N_NODES, PAYW), lambda i: (0, 0, 0)),
                pl.BlockSpec((NC, N_NODES, PAYW), lambda i: (0, 1, 0)),
                pl.BlockSpec((1, F), lambda i: (0, 0))],
      out_specs=pl.BlockSpec((N_NODES, F), lambda i: (0, 0)),
      out_shape=jax.ShapeDtypeStruct((N_NODES, F), jnp.float32),
  )(spart, spart, bias2)


def kernel(feat, efeat, W_attn, b_attn, W_e1, b_e1, W_e2, b_e2, bias,
           edge_index, etype):
  src = edge_index[0].astype(jnp.int32)
  dst = edge_index[1].astype(jnp.int32)
  et = etype.astype(jnp.int32)
  hsrc, zdst = _sc_gather(feat, src, dst)
  wcat = jnp.concatenate(
      [W_e1.reshape(F * F, F), W_e2.reshape(F * F, F)], axis=1)
  bcat = jnp.concatenate([b_e1.reshape(F, F), b_e2.reshape(F, F)], axis=1)
  was = W_attn[0:F, 0].reshape(1, F)
  wad = W_attn[F:2 * F, 0].reshape(1, F)
  ba = b_attn.reshape(1, 1)
  payload = _tc_dense(hsrc, zdst, efeat, et.reshape(-1, 1), was, wad, ba,
                      wcat, bcat)
  spart = _sc_scatter(payload, dst, et)
  return _tc_finalize(spart, bias.reshape(1, F))
